# baseline (speedup 1.0000x reference)
import functools

import jax
import jax.numpy as jnp
from jax.experimental import pallas as pl

E = 8
H = 256
NB = 8

_NEG = -1e30


def _top2_gates(logits):
    T = logits.shape[0]
    iota = jax.lax.broadcasted_iota(jnp.int32, (T, E), 1)
    max1 = jnp.max(logits, axis=-1, keepdims=True)
    idx1 = jnp.min(jnp.where(logits == max1, iota, E), axis=-1, keepdims=True)
    masked = jnp.where(iota == idx1, _NEG, logits)
    max2 = jnp.max(masked, axis=-1, keepdims=True)
    idx2 = jnp.min(jnp.where(masked == max2, iota, E), axis=-1, keepdims=True)
    e2 = jnp.exp(max2 - max1)
    denom = 1.0 + e2
    gk1 = 1.0 / denom
    gk2 = e2 / denom
    return jnp.where(iota == idx1, gk1, 0.0) + jnp.where(iota == idx2, gk2, 0.0)


def _moe_kernel(xf_ref, wr0_ref, br0_ref, we0_ref, be0_ref, wres0_ref,
                bres0_ref, wr1_ref, br1_ref, we1_ref, be1_ref,
                wr2_ref, br2_ref, we2_ref, be2_ref, wres2_ref, bres2_ref,
                wfct_ref, bfc_ref,
                out_ref, g0_ref, g1_ref, g2_ref, *, L):
    xv = xf_ref[:]

    logits0 = xv * wr0_ref[:] + br0_ref[:]
    g0 = _top2_gates(logits0)
    f1 = (xv * (jnp.dot(g0, we0_ref[:], preferred_element_type=jnp.float32, precision=jax.lax.Precision.HIGHEST)
                + wres0_ref[:])
          + jnp.dot(g0, be0_ref[:], preferred_element_type=jnp.float32, precision=jax.lax.Precision.HIGHEST)
          + bres0_ref[:])

    logits1 = jnp.dot(f1, wr1_ref[:], preferred_element_type=jnp.float32,
                      precision=jax.lax.Precision.DEFAULT)
    logits1 = logits1 + br1_ref[:]
    g1 = _top2_gates(logits1)
    acc = jnp.dot(g1, be1_ref[:], preferred_element_type=jnp.float32, precision=jax.lax.Precision.HIGHEST) + f1
    for e in range(E):
        pe = jnp.dot(f1, we1_ref[e], preferred_element_type=jnp.float32,
                     precision=jax.lax.Precision.DEFAULT)
        acc = acc + g1[:, e:e + 1] * pe
    f2 = acc

    logits2 = jnp.dot(f2, wr2_ref[:], preferred_element_type=jnp.float32,
                      precision=jax.lax.Precision.DEFAULT)
    logits2 = logits2 + br2_ref[:]
    g2 = _top2_gates(logits2)
    mix = jnp.dot(g2, we2_ref[:], preferred_element_type=jnp.float32, precision=jax.lax.Precision.HIGHEST)
    prod = f2 * (mix + wres2_ref[:])
    tok_sum = jnp.sum(prod.reshape(NB, L, H), axis=-1)
    sc = (g2 * be2_ref[:]).reshape(NB, L, E)
    f3 = tok_sum + jnp.sum(sc, axis=-1) + bres2_ref[0, 0]

    out_ref[:] = (jnp.dot(f3, wfct_ref[:], preferred_element_type=jnp.float32, precision=jax.lax.Precision.HIGHEST)
                  + bfc_ref[:])

    g0_ref[:] = g0.reshape(NB, L, E)
    g1_ref[:] = g1.reshape(NB, L, E)
    g2_ref[:] = g2.reshape(NB, L, E)


def kernel(x, Wr0, br0, We0, be0, Wres0, bres0, Wr1, br1, We1, be1,
           Wr2, br2, We2, be2, Wres2, bres2, Wfc, bfc):
    B, _, L = x.shape
    F = Wfc.shape[0]
    xf = x.reshape(B * L, 1)

    full = lambda shape: pl.BlockSpec(shape, lambda i: (0,) * len(shape))
    grid = (B // NB,)

    out2d, g0, g1, g2 = pl.pallas_call(
        functools.partial(_moe_kernel, L=L),
        grid=grid,
        in_specs=[
            pl.BlockSpec((NB * L, 1), lambda i: (i, 0)),
            full((1, E)),
            full((1, E)),
            full((E, H)),
            full((E, H)),
            full((1, H)),
            full((1, H)),
            full((H, E)),
            full((1, E)),
            full((E, H, H)),
            full((E, H)),
            full((H, E)),
            full((1, E)),
            full((E, H)),
            full((1, E)),
            full((1, H)),
            full((1, 1)),
            full((L, F)),
            full((1, F)),
        ],
        out_specs=[
            pl.BlockSpec((NB, F), lambda i: (i, 0)),
            pl.BlockSpec((NB, L, E), lambda i: (i, 0, 0)),
            pl.BlockSpec((NB, L, E), lambda i: (i, 0, 0)),
            pl.BlockSpec((NB, L, E), lambda i: (i, 0, 0)),
        ],
        out_shape=[
            jax.ShapeDtypeStruct((B, F), jnp.float32),
            jax.ShapeDtypeStruct((B, L, E), jnp.float32),
            jax.ShapeDtypeStruct((B, L, E), jnp.float32),
            jax.ShapeDtypeStruct((B, L, E), jnp.float32),
        ],
    )(
        xf,
        Wr0.reshape(1, E), br0.reshape(1, E),
        We0[:, 0, :], be0,
        Wres0[:, 0].reshape(1, H), bres0.reshape(1, H),
        Wr1, br1.reshape(1, E),
        We1, be1,
        Wr2, br2.reshape(1, E),
        We2[:, :, 0], be2[:, 0].reshape(1, E),
        Wres2.reshape(1, H), bres2.reshape(1, 1),
        Wfc.T, bfc.reshape(1, F),
    )
    return (out2d.reshape(B, 1, F), g0, g1, g2)


# speedup vs baseline: 1.2459x; 1.2459x over previous
import functools

import jax
import jax.numpy as jnp
from jax.experimental import pallas as pl

E = 8
H = 256
NB = 8

_DEFAULT = jax.lax.Precision.DEFAULT


def _split3(a):
    a1 = a.astype(jnp.bfloat16)
    r = a - a1.astype(jnp.float32)
    a2 = r.astype(jnp.bfloat16)
    a3 = (r - a2.astype(jnp.float32)).astype(jnp.bfloat16)
    return a1, a2, a3


def _dot_f32(a, b):
    a1, a2, a3 = _split3(a)
    b1, b2, b3 = _split3(b)
    d = lambda x, y: jnp.dot(x, y, preferred_element_type=jnp.float32)
    return ((d(a1, b3) + d(a2, b2) + d(a3, b1))
            + (d(a1, b2) + d(a2, b1))) + d(a1, b1)


def _dot_high(a, b):
    a_hi = a.astype(jnp.bfloat16)
    a_lo = (a - a_hi.astype(jnp.float32)).astype(jnp.bfloat16)
    b_hi = b.astype(jnp.bfloat16)
    b_lo = (b - b_hi.astype(jnp.float32)).astype(jnp.bfloat16)
    d = lambda x, y: jnp.dot(x, y, preferred_element_type=jnp.float32)
    return d(a_hi, b_hi) + (d(a_hi, b_lo) + d(a_lo, b_hi))


def _key_val(k):
    kb = k & ~7
    bits = jnp.where(kb < 0, kb ^ 0x7FFFFFFF, kb)
    return jax.lax.bitcast_convert_type(bits, jnp.float32)


def _top2_gates(logits, rev_iota):
    bits = jax.lax.bitcast_convert_type(logits, jnp.int32)
    mono = jnp.where(bits < 0, bits ^ 0x7FFFFFFF, bits)
    key = (mono & ~7) | rev_iota
    k1 = jnp.max(key, axis=-1, keepdims=True)
    m1 = key == k1
    masked = jnp.where(m1, jnp.int32(-2**31), key)
    k2 = jnp.max(masked, axis=-1, keepdims=True)
    m2 = masked == k2
    e2 = jnp.exp(_key_val(k2) - _key_val(k1))
    gk1 = 1.0 / (1.0 + e2)
    gk2 = 1.0 - gk1
    return jnp.where(m1, gk1, 0.0) + jnp.where(m2, gk2, 0.0)


def _moe_kernel(xf_ref, wr0_ref, br0_ref, w0full_ref,
                wr1_ref, br1_ref, we1_ref, be1_ref,
                wr2_ref, br2_ref, w2cat_ref, be2_ref, bres2_ref,
                wfct_ref, bfc_ref,
                out_ref, g0_ref, g1_ref, g2_ref, *, L):
    T = NB * L
    xv = xf_ref[:]
    rev_iota = jax.lax.broadcasted_iota(jnp.int32, (T, E), 1) ^ 7

    logits0 = xv * wr0_ref[:] + br0_ref[:]
    g0 = _top2_gates(logits0, rev_iota)
    g0x = jnp.concatenate(
        [xv * g0, g0, xv, jnp.ones((T, 1), jnp.float32)], axis=1)
    f1 = _dot_f32(g0x, w0full_ref[:])

    logits1 = jnp.dot(f1, wr1_ref[:], preferred_element_type=jnp.float32,
                      precision=_DEFAULT) + br1_ref[:]
    g1 = _top2_gates(logits1, rev_iota)
    acc = None
    for e in range(E):
        pe = jnp.dot(f1, we1_ref[e], preferred_element_type=jnp.float32,
                     precision=_DEFAULT)
        te = g1[:, e:e + 1] * (pe + be1_ref[e:e + 1, :])
        acc = te if acc is None else acc + te
    f2 = acc + f1

    logits2 = jnp.dot(f2, wr2_ref[:], preferred_element_type=jnp.float32,
                      precision=_DEFAULT) + br2_ref[:]
    g2 = _top2_gates(logits2, rev_iota)
    qq = jnp.dot(f2, w2cat_ref[:], preferred_element_type=jnp.float32,
                 precision=_DEFAULT)
    ft = jnp.concatenate(
        [g2 * (qq[:, :E] + be2_ref[:]), qq[:, E:]], axis=1)
    f3 = jnp.sum(ft.reshape(NB, L, E + 1), axis=-1) + bres2_ref[0, 0]

    out_ref[:] = _dot_high(f3, wfct_ref[:]) + bfc_ref[:]

    g0_ref[:] = g0.reshape(NB, L, E)
    g1_ref[:] = g1.reshape(NB, L, E)
    g2_ref[:] = g2.reshape(NB, L, E)


def kernel(x, Wr0, br0, We0, be0, Wres0, bres0, Wr1, br1, We1, be1,
           Wr2, br2, We2, be2, Wres2, bres2, Wfc, bfc):
    B, _, L = x.shape
    F = Wfc.shape[0]
    xf = x.reshape(B * L, 1)
    w0full = jnp.concatenate(
        [We0[:, 0, :], be0, Wres0[:, 0].reshape(1, H), bres0.reshape(1, H)],
        axis=0)
    w2cat = jnp.concatenate(
        [We2[:, :, 0].T, Wres2.reshape(H, 1)], axis=1)

    full = lambda shape: pl.BlockSpec(shape, lambda i: (0,) * len(shape))
    grid = (B // NB,)

    out2d, g0, g1, g2 = pl.pallas_call(
        functools.partial(_moe_kernel, L=L),
        grid=grid,
        in_specs=[
            pl.BlockSpec((NB * L, 1), lambda i: (i, 0)),
            full((1, E)),
            full((1, E)),
            full((2 * E + 2, H)),
            full((H, E)),
            full((1, E)),
            full((E, H, H)),
            full((E, H)),
            full((H, E)),
            full((1, E)),
            full((H, E + 1)),
            full((1, E)),
            full((1, 1)),
            full((L, F)),
            full((1, F)),
        ],
        out_specs=[
            pl.BlockSpec((NB, F), lambda i: (i, 0)),
            pl.BlockSpec((NB, L, E), lambda i: (i, 0, 0)),
            pl.BlockSpec((NB, L, E), lambda i: (i, 0, 0)),
            pl.BlockSpec((NB, L, E), lambda i: (i, 0, 0)),
        ],
        out_shape=[
            jax.ShapeDtypeStruct((B, F), jnp.float32),
            jax.ShapeDtypeStruct((B, L, E), jnp.float32),
            jax.ShapeDtypeStruct((B, L, E), jnp.float32),
            jax.ShapeDtypeStruct((B, L, E), jnp.float32),
        ],
    )(
        xf,
        Wr0.reshape(1, E), br0.reshape(1, E),
        w0full,
        Wr1, br1.reshape(1, E),
        We1, be1,
        Wr2, br2.reshape(1, E),
        w2cat, be2[:, 0].reshape(1, E), bres2.reshape(1, 1),
        Wfc.T, bfc.reshape(1, F),
    )
    return (out2d.reshape(B, 1, F), g0, g1, g2)
